# baseline (speedup 1.0000x reference)
import functools

import jax
import jax.numpy as jnp
from jax import lax
from jax.experimental import pallas as pl
from jax.experimental.pallas import tpu as pltpu
from jax.experimental.pallas import tpu_sc as plsc

N = 10000
D = 128
E = 320000
NC, NS = 2, 16
NW = NC * NS
CH = 128
NGT = 160
NG0, NG1 = 48, 112
NCH = NGT * NS
EPAD = NCH * CH
NPAD = 10240
RT = NPAD // NS

SRC0, DST0, SRC1, DST1 = 0, 1, 2, 3


def _fill_rows(ref, nrows, value):
    vec = jnp.full((16,), value, jnp.float32)

    def body(i, _):
        for j in range(ref.shape[1] // 16):
            ref[i, pl.ds(j * 16, 16)] = vec
        return 0

    lax.fori_loop(0, nrows, body, 0)


def _decode(sd_all, g, idx_v, src_row, dst_row):
    for j in range(CH // 16):
        v = sd_all[g, pl.ds(j * 16, 16)]
        idx_v[src_row, pl.ds(j * 16, 16)] = lax.bitwise_and(v, 16383)
        idx_v[dst_row, pl.ds(j * 16, 16)] = lax.shift_right_logical(v, 14)


def _agg_pipeline(ng, x, accum, sd_all, idx_v, rows, gsems, ssems):
    src_rows = (SRC0, SRC1)
    dst_rows = (DST0, DST1)

    def scatter_desc(b):
        return pltpu.make_async_copy(
            rows[b], accum.at[idx_v.at[dst_rows[b]]], ssems[b])

    def gather_desc(b):
        return pltpu.make_async_copy(
            x.at[idx_v.at[src_rows[b]]], rows[b], gsems[b])

    _decode(sd_all, 0, idx_v, SRC0, DST0)
    gather_desc(0).start()

    def pair(k, _):
        for b in (0, 1):
            g = k * 2 + b
            nb = 1 - b
            gn = g + 1

            @pl.when(gn < ng)
            def _prefetch():
                @pl.when(gn >= 2)
                def _drain():
                    scatter_desc(nb).wait()

                _decode(sd_all, gn, idx_v, src_rows[nb], dst_rows[nb])
                gather_desc(nb).start()

            gather_desc(b).wait()
            scatter_desc(b).start(add=True)

        return 0

    lax.fori_loop(0, ng // 2, pair, 0)

    scatter_desc(0).wait()
    scatter_desc(1).wait()


def _sc_agg_body(x, sdg, part_out, sd_all, idx_v, rows_a, rows_b,
                 accum, gsem_a, gsem_b, ssem_a, ssem_b):
    c = lax.axis_index("c")
    s = lax.axis_index("s")
    base = s * RT

    _fill_rows(rows_a, CH, 0.0)

    def zacc(j, _):
        pltpu.sync_copy(rows_a, accum.at[pl.ds(base + j * CH, CH)])
        return 0

    lax.fori_loop(0, RT // CH, zacc, 0)

    rows = (rows_a, rows_b)
    gsems = (gsem_a, gsem_b)
    ssems = (ssem_a, ssem_b)

    @pl.when(c == 0)
    def _core0():
        pltpu.sync_copy(sdg.at[pl.ds(s * NG0, NG0)],
                        sd_all.at[pl.ds(0, NG0)])
        plsc.subcore_barrier()
        _agg_pipeline(NG0, x, accum, sd_all, idx_v, rows, gsems, ssems)

    @pl.when(c == 1)
    def _core1():
        pltpu.sync_copy(sdg.at[pl.ds(NS * NG0 + s * NG1, NG1)],
                        sd_all.at[pl.ds(0, NG1)])
        plsc.subcore_barrier()
        _agg_pipeline(NG1, x, accum, sd_all, idx_v, rows, gsems, ssems)

    plsc.subcore_barrier()

    pltpu.sync_copy(accum.at[pl.ds(base, RT)],
                    part_out.at[c, pl.ds(base, RT)])


def _sc_deg_body(sdg, deg_out, sd_all, idx_v, ones_v, deg_acc,
                 ssem_a, ssem_b):
    c = lax.axis_index("c")
    s = lax.axis_index("s")
    base = s * RT

    _fill_rows(ones_v, CH, 0.0)

    def zdeg(j, _):
        pltpu.sync_copy(ones_v, deg_acc.at[pl.ds(base + j * CH, CH)])
        return 0

    lax.fori_loop(0, RT // CH, zdeg, 0)

    _fill_rows(ones_v, CH, 1.0)

    ssems = (ssem_a, ssem_b)
    src_rows = (SRC0, SRC1)
    dst_rows = (DST0, DST1)

    def scatter_desc(b):
        return pltpu.make_async_copy(
            ones_v, deg_acc.at[idx_v.at[dst_rows[b]]], ssems[b])

    def run(ng):
        def pair(k, _):
            for b in (0, 1):
                g = k * 2 + b

                @pl.when(g >= 2)
                def _drain():
                    scatter_desc(b).wait()

                _decode(sd_all, g, idx_v, src_rows[b], dst_rows[b])
                scatter_desc(b).start(add=True)

            return 0

        lax.fori_loop(0, ng // 2, pair, 0)
        scatter_desc(0).wait()
        scatter_desc(1).wait()

    @pl.when(c == 0)
    def _core0():
        pltpu.sync_copy(sdg.at[pl.ds(s * NG0, NG0)],
                        sd_all.at[pl.ds(0, NG0)])
        plsc.subcore_barrier()
        run(NG0)

    @pl.when(c == 1)
    def _core1():
        pltpu.sync_copy(sdg.at[pl.ds(NS * NG0 + s * NG1, NG1)],
                        sd_all.at[pl.ds(0, NG1)])
        plsc.subcore_barrier()
        run(NG1)

    plsc.subcore_barrier()

    pltpu.sync_copy(deg_acc.at[pl.ds(base, RT)],
                    deg_out.at[c, pl.ds(base, RT)])


@functools.lru_cache(maxsize=1)
def _sc_kernels():
    mesh = plsc.VectorSubcoreMesh(
        core_axis_name="c", subcore_axis_name="s",
        num_cores=NC, num_subcores=NS)
    agg = pl.kernel(
        _sc_agg_body,
        out_type=jax.ShapeDtypeStruct((NC, NPAD, D), jnp.float32),
        mesh=mesh,
        scratch_types=[
            pltpu.VMEM((max(NG0, NG1), CH), jnp.int32),
            pltpu.VMEM((8, CH), jnp.int32),
            pltpu.VMEM((CH, D), jnp.float32),
            pltpu.VMEM((CH, D), jnp.float32),
            pltpu.VMEM_SHARED((NPAD, D), jnp.float32),
            pltpu.SemaphoreType.DMA,
            pltpu.SemaphoreType.DMA,
            pltpu.SemaphoreType.DMA,
            pltpu.SemaphoreType.DMA,
        ],
    )
    deg = pl.kernel(
        _sc_deg_body,
        out_type=jax.ShapeDtypeStruct((NC, NPAD, D), jnp.float32),
        mesh=mesh,
        scratch_types=[
            pltpu.VMEM((max(NG0, NG1), CH), jnp.int32),
            pltpu.VMEM((8, CH), jnp.int32),
            pltpu.VMEM((CH, D), jnp.float32),
            pltpu.VMEM_SHARED((NPAD, D), jnp.float32),
            pltpu.SemaphoreType.DMA,
            pltpu.SemaphoreType.DMA,
        ],
    )
    return agg, deg


BN = 1024
GRID = NPAD // BN


def _tc_body(x_ref, p_ref, d_ref, wn_ref, ws_ref, b_ref, o_ref):
    p = p_ref[0] + p_ref[1]
    dsum = d_ref[0] + d_ref[1]
    deg = dsum[:, 0:1]
    mean = p * (1.0 / jnp.maximum(deg, 1.0))
    dn = (((1,), (1,)), ((), ()))
    hn = lax.dot_general(mean, wn_ref[...], dn,
                         preferred_element_type=jnp.float32)
    hs = lax.dot_general(x_ref[...], ws_ref[...], dn,
                         preferred_element_type=jnp.float32)
    o_ref[...] = hs + hn + b_ref[...]


_tc_layer = pl.pallas_call(
    _tc_body,
    grid=(GRID,),
    in_specs=[
        pl.BlockSpec((BN, D), lambda i: (i, 0)),
        pl.BlockSpec((NC, BN, D), lambda i: (0, i, 0)),
        pl.BlockSpec((NC, BN, D), lambda i: (0, i, 0)),
        pl.BlockSpec((D, D), lambda i: (0, 0)),
        pl.BlockSpec((D, D), lambda i: (0, 0)),
        pl.BlockSpec((1, D), lambda i: (0, 0)),
    ],
    out_specs=pl.BlockSpec((BN, D), lambda i: (i, 0)),
    out_shape=jax.ShapeDtypeStruct((NPAD, D), jnp.float32),
)


def kernel(h, edge_index, W_neigh1, W_self1, b_self1,
           W_neigh2, W_self2, b_self2):
    src = edge_index[0]
    dst = edge_index[1]
    pad = EPAD - E
    pad_dst = N + jnp.arange(pad, dtype=jnp.int32) % (NPAD - N)
    sd = jnp.concatenate([
        jnp.bitwise_or(src, jnp.left_shift(dst, 14)),
        jnp.left_shift(pad_dst, 14),
    ]).reshape(NCH, CH)
    x0 = jnp.pad(h, ((0, NPAD - N), (0, 0)))
    b1 = b_self1.reshape(1, D)
    b2 = b_self2.reshape(1, D)

    sc_agg, sc_deg = _sc_kernels()
    degp = sc_deg(sd)
    part1 = sc_agg(x0, sd)
    h1 = _tc_layer(x0, part1, degp, W_neigh1, W_self1, b1)
    part2 = sc_agg(h1, sd)
    h2 = _tc_layer(h1, part2, degp, W_neigh2, W_self2, b2)
    return h2[:N]


# speedup vs baseline: 1.4386x; 1.4386x over previous
import functools

import jax
import jax.numpy as jnp
from jax import lax
from jax.experimental import pallas as pl
from jax.experimental.pallas import tpu as pltpu
from jax.experimental.pallas import tpu_sc as plsc

N = 10000
D = 128
E = 320000
NC, NS = 2, 16
NW = NC * NS
CH = 128
NG = -(-E // (NW * CH))
EPT = NG * CH
EPAD = EPT * NW
NPAD = 10240
RT = NPAD // NS

SRC0, DST0, SRC1, DST1, SD0, SD1 = 0, 1, 2, 3, 4, 5


def _fill_rows(ref, nrows, value):
    vec = jnp.full((16,), value, jnp.float32)

    def body(i, _):
        for j in range(ref.shape[1] // 16):
            ref[i, pl.ds(j * 16, 16)] = vec
        return 0

    lax.fori_loop(0, nrows, body, 0)


def _decode(sd_all, g, idx_v, src_row, dst_row):
    for j in range(CH // 16):
        v = sd_all[g, pl.ds(j * 16, 16)]
        idx_v[src_row, pl.ds(j * 16, 16)] = lax.bitwise_and(v, 16383)
        idx_v[dst_row, pl.ds(j * 16, 16)] = lax.shift_right_logical(v, 14)


def _sc_agg_body(x, sdg, part_out, sd_all, idx_v, rows_a, rows_b,
                 accum, gsem_a, gsem_b, ssem_a, ssem_b):
    c = lax.axis_index("c")
    s = lax.axis_index("s")
    wid = s * NC + c
    base = s * RT

    _fill_rows(rows_a, CH, 0.0)

    def zacc(j, _):
        pltpu.sync_copy(rows_a, accum.at[pl.ds(base + j * CH, CH)])
        return 0

    lax.fori_loop(0, RT // CH, zacc, 0)

    pltpu.sync_copy(sdg.at[wid], sd_all)

    plsc.subcore_barrier()

    rows = (rows_a, rows_b)
    gsems = (gsem_a, gsem_b)
    ssems = (ssem_a, ssem_b)
    src_rows = (SRC0, SRC1)
    dst_rows = (DST0, DST1)

    def scatter_desc(b):
        return pltpu.make_async_copy(
            rows[b], accum.at[idx_v.at[dst_rows[b]]], ssems[b])

    def gather_desc(b):
        return pltpu.make_async_copy(
            x.at[idx_v.at[src_rows[b]]], rows[b], gsems[b])

    _decode(sd_all, 0, idx_v, SRC0, DST0)
    gather_desc(0).start()

    def pair(k, _):
        for b in (0, 1):
            g = k * 2 + b
            nb = 1 - b

            @pl.when(g < NG)
            def _work():
                gn = g + 1

                @pl.when(gn < NG)
                def _prefetch():
                    @pl.when(gn >= 2)
                    def _drain():
                        scatter_desc(nb).wait()

                    _decode(sd_all, gn, idx_v, src_rows[nb], dst_rows[nb])
                    gather_desc(nb).start()

                pltpu.make_async_copy(
                    x.at[idx_v.at[src_rows[b]]], rows[b], gsems[b]).wait()
                scatter_desc(b).start(add=True)

        return 0

    lax.fori_loop(0, (NG + 1) // 2, pair, 0)

    scatter_desc((NG - 2) % 2).wait()
    scatter_desc((NG - 1) % 2).wait()

    plsc.subcore_barrier()

    pltpu.sync_copy(accum.at[pl.ds(base, RT)],
                    part_out.at[c, pl.ds(base, RT)])


def _sc_deg_body(sdg, deg_out, sd_all, idx_v, ones_v, deg_acc,
                 ssem_a, ssem_b):
    c = lax.axis_index("c")
    s = lax.axis_index("s")
    wid = s * NC + c
    base = s * RT

    _fill_rows(ones_v, CH, 0.0)

    def zdeg(j, _):
        pltpu.sync_copy(ones_v, deg_acc.at[pl.ds(base + j * CH, CH)])
        return 0

    lax.fori_loop(0, RT // CH, zdeg, 0)

    _fill_rows(ones_v, CH, 1.0)

    pltpu.sync_copy(sdg.at[wid], sd_all)

    plsc.subcore_barrier()

    ssems = (ssem_a, ssem_b)
    dst_rows = (DST0, DST1)

    def scatter_desc(b):
        return pltpu.make_async_copy(
            ones_v, deg_acc.at[idx_v.at[dst_rows[b]]], ssems[b])

    src_rows = (SRC0, SRC1)

    def pair(k, _):
        for b in (0, 1):
            g = k * 2 + b

            @pl.when(g < NG)
            def _work():
                @pl.when(g >= 2)
                def _drain():
                    scatter_desc(b).wait()

                _decode(sd_all, g, idx_v, src_rows[b], dst_rows[b])
                scatter_desc(b).start(add=True)

        return 0

    lax.fori_loop(0, (NG + 1) // 2, pair, 0)

    scatter_desc((NG - 2) % 2).wait()
    scatter_desc((NG - 1) % 2).wait()

    plsc.subcore_barrier()

    pltpu.sync_copy(deg_acc.at[pl.ds(base, RT)],
                    deg_out.at[c, pl.ds(base, RT)])


@functools.lru_cache(maxsize=1)
def _sc_kernels():
    mesh = plsc.VectorSubcoreMesh(
        core_axis_name="c", subcore_axis_name="s",
        num_cores=NC, num_subcores=NS)
    agg = pl.kernel(
        _sc_agg_body,
        out_type=jax.ShapeDtypeStruct((NC, NPAD, D), jnp.float32),
        mesh=mesh,
        scratch_types=[
            pltpu.VMEM((NG, CH), jnp.int32),
            pltpu.VMEM((8, CH), jnp.int32),
            pltpu.VMEM((CH, D), jnp.float32),
            pltpu.VMEM((CH, D), jnp.float32),
            pltpu.VMEM_SHARED((NPAD, D), jnp.float32),
            pltpu.SemaphoreType.DMA,
            pltpu.SemaphoreType.DMA,
            pltpu.SemaphoreType.DMA,
            pltpu.SemaphoreType.DMA,
        ],
    )
    deg = pl.kernel(
        _sc_deg_body,
        out_type=jax.ShapeDtypeStruct((NC, NPAD, D), jnp.float32),
        mesh=mesh,
        scratch_types=[
            pltpu.VMEM((NG, CH), jnp.int32),
            pltpu.VMEM((8, CH), jnp.int32),
            pltpu.VMEM((CH, D), jnp.float32),
            pltpu.VMEM_SHARED((NPAD, D), jnp.float32),
            pltpu.SemaphoreType.DMA,
            pltpu.SemaphoreType.DMA,
        ],
    )
    return agg, deg


BN = 1024
GRID = NPAD // BN


def _tc_body(x_ref, p_ref, d_ref, wn_ref, ws_ref, b_ref, o_ref):
    p = p_ref[0] + p_ref[1]
    dsum = d_ref[0] + d_ref[1]
    deg = dsum[:, 0:1]
    mean = p * (1.0 / jnp.maximum(deg, 1.0))
    dn = (((1,), (1,)), ((), ()))
    hn = lax.dot_general(mean, wn_ref[...], dn,
                         preferred_element_type=jnp.float32)
    hs = lax.dot_general(x_ref[...], ws_ref[...], dn,
                         preferred_element_type=jnp.float32)
    o_ref[...] = hs + hn + b_ref[...]


_tc_layer = pl.pallas_call(
    _tc_body,
    grid=(GRID,),
    in_specs=[
        pl.BlockSpec((BN, D), lambda i: (i, 0)),
        pl.BlockSpec((NC, BN, D), lambda i: (0, i, 0)),
        pl.BlockSpec((NC, BN, D), lambda i: (0, i, 0)),
        pl.BlockSpec((D, D), lambda i: (0, 0)),
        pl.BlockSpec((D, D), lambda i: (0, 0)),
        pl.BlockSpec((1, D), lambda i: (0, 0)),
    ],
    out_specs=pl.BlockSpec((BN, D), lambda i: (i, 0)),
    out_shape=jax.ShapeDtypeStruct((NPAD, D), jnp.float32),
)


def kernel(h, edge_index, W_neigh1, W_self1, b_self1,
           W_neigh2, W_self2, b_self2):
    src = edge_index[0]
    dst = edge_index[1]
    pad = EPAD - E
    pad_dst = N + jnp.arange(pad, dtype=jnp.int32) % (NPAD - N)
    sd = jnp.concatenate([
        jnp.bitwise_or(src, jnp.left_shift(dst, 14)),
        jnp.left_shift(pad_dst, 14),
    ]).reshape(NW, NG, CH)
    x0 = jnp.pad(h, ((0, NPAD - N), (0, 0)))
    b1 = b_self1.reshape(1, D)
    b2 = b_self2.reshape(1, D)

    sc_agg, sc_deg = _sc_kernels()
    degp = sc_deg(sd)
    part1 = sc_agg(x0, sd)
    h1 = _tc_layer(x0, part1, degp, W_neigh1, W_self1, b1)
    part2 = sc_agg(h1, sd)
    h2 = _tc_layer(h1, part2, degp, W_neigh2, W_self2, b2)
    return h2[:N]


# speedup vs baseline: 1.4421x; 1.0025x over previous
import functools

import jax
import jax.numpy as jnp
from jax import lax
from jax.experimental import pallas as pl
from jax.experimental.pallas import tpu as pltpu
from jax.experimental.pallas import tpu_sc as plsc

N = 10000
D = 128
E = 320000
NC, NS = 2, 16
NW = NC * NS
CH = 128
NG = -(-E // (NW * CH))
EPT = NG * CH
EPAD = EPT * NW
NPAD = 10240
RT = NPAD // NS

SRC0, DST0, SRC1, DST1, SD0, SD1 = 0, 1, 2, 3, 4, 5


def _fill_rows(ref, nrows, value):
    vec = jnp.full((16,), value, jnp.float32)

    def body(i, _):
        for j in range(ref.shape[1] // 16):
            ref[i, pl.ds(j * 16, 16)] = vec
        return 0

    lax.fori_loop(0, nrows, body, 0)


def _decode(sd_all, g, idx_v, src_row, dst_row):
    for j in range(CH // 16):
        v = sd_all[g, pl.ds(j * 16, 16)]
        idx_v[src_row, pl.ds(j * 16, 16)] = lax.bitwise_and(v, 16383)
        idx_v[dst_row, pl.ds(j * 16, 16)] = lax.shift_right_logical(v, 14)


def _sc_agg_body(x, sdg, part_out, sd_all, idx_v, rows_a, rows_b,
                 accum, gsem_a, gsem_b, ssem_a, ssem_b):
    c = lax.axis_index("c")
    s = lax.axis_index("s")
    wid = s * NC + c
    base = s * RT

    _fill_rows(rows_a, CH, 0.0)

    def zacc(j, _):
        pltpu.sync_copy(rows_a, accum.at[pl.ds(base + j * CH, CH)])
        return 0

    lax.fori_loop(0, RT // CH, zacc, 0)

    pltpu.sync_copy(sdg.at[wid], sd_all)

    plsc.subcore_barrier()

    rows = (rows_a, rows_b)
    gsems = (gsem_a, gsem_b)
    ssems = (ssem_a, ssem_b)
    src_rows = (SRC0, SRC1)
    dst_rows = (DST0, DST1)

    def scatter_desc(b):
        return pltpu.make_async_copy(
            rows[b], accum.at[idx_v.at[dst_rows[b]]], ssems[b])

    def gather_desc(b):
        return pltpu.make_async_copy(
            x.at[idx_v.at[src_rows[b]]], rows[b], gsems[b])

    _decode(sd_all, 0, idx_v, SRC0, DST0)
    gather_desc(0).start(priority=1)

    def pair(k, _):
        for b in (0, 1):
            g = k * 2 + b
            nb = 1 - b

            @pl.when(g < NG)
            def _work():
                gn = g + 1

                @pl.when(gn < NG)
                def _prefetch():
                    @pl.when(gn >= 2)
                    def _drain():
                        scatter_desc(nb).wait()

                    _decode(sd_all, gn, idx_v, src_rows[nb], dst_rows[nb])
                    gather_desc(nb).start(priority=1)

                pltpu.make_async_copy(
                    x.at[idx_v.at[src_rows[b]]], rows[b], gsems[b]).wait()
                scatter_desc(b).start(add=True)

        return 0

    lax.fori_loop(0, (NG + 1) // 2, pair, 0)

    scatter_desc((NG - 2) % 2).wait()
    scatter_desc((NG - 1) % 2).wait()

    plsc.subcore_barrier()

    pltpu.sync_copy(accum.at[pl.ds(base, RT)],
                    part_out.at[c, pl.ds(base, RT)])


def _sc_deg_body(sdg, deg_out, sd_all, idx_v, ones_v, deg_acc,
                 ssem_a, ssem_b):
    c = lax.axis_index("c")
    s = lax.axis_index("s")
    wid = s * NC + c
    base = s * RT

    _fill_rows(ones_v, CH, 0.0)

    def zdeg(j, _):
        pltpu.sync_copy(ones_v, deg_acc.at[pl.ds(base + j * CH, CH)])
        return 0

    lax.fori_loop(0, RT // CH, zdeg, 0)

    _fill_rows(ones_v, CH, 1.0)

    pltpu.sync_copy(sdg.at[wid], sd_all)

    plsc.subcore_barrier()

    ssems = (ssem_a, ssem_b)
    dst_rows = (DST0, DST1)

    def scatter_desc(b):
        return pltpu.make_async_copy(
            ones_v, deg_acc.at[idx_v.at[dst_rows[b]]], ssems[b])

    src_rows = (SRC0, SRC1)

    def pair(k, _):
        for b in (0, 1):
            g = k * 2 + b

            @pl.when(g < NG)
            def _work():
                @pl.when(g >= 2)
                def _drain():
                    scatter_desc(b).wait()

                _decode(sd_all, g, idx_v, src_rows[b], dst_rows[b])
                scatter_desc(b).start(add=True)

        return 0

    lax.fori_loop(0, (NG + 1) // 2, pair, 0)

    scatter_desc((NG - 2) % 2).wait()
    scatter_desc((NG - 1) % 2).wait()

    plsc.subcore_barrier()

    pltpu.sync_copy(deg_acc.at[pl.ds(base, RT)],
                    deg_out.at[c, pl.ds(base, RT)])


@functools.lru_cache(maxsize=1)
def _sc_kernels():
    mesh = plsc.VectorSubcoreMesh(
        core_axis_name="c", subcore_axis_name="s",
        num_cores=NC, num_subcores=NS)
    agg = pl.kernel(
        _sc_agg_body,
        out_type=jax.ShapeDtypeStruct((NC, NPAD, D), jnp.float32),
        mesh=mesh,
        scratch_types=[
            pltpu.VMEM((NG, CH), jnp.int32),
            pltpu.VMEM((8, CH), jnp.int32),
            pltpu.VMEM((CH, D), jnp.float32),
            pltpu.VMEM((CH, D), jnp.float32),
            pltpu.VMEM_SHARED((NPAD, D), jnp.float32),
            pltpu.SemaphoreType.DMA,
            pltpu.SemaphoreType.DMA,
            pltpu.SemaphoreType.DMA,
            pltpu.SemaphoreType.DMA,
        ],
    )
    deg = pl.kernel(
        _sc_deg_body,
        out_type=jax.ShapeDtypeStruct((NC, NPAD, D), jnp.float32),
        mesh=mesh,
        scratch_types=[
            pltpu.VMEM((NG, CH), jnp.int32),
            pltpu.VMEM((8, CH), jnp.int32),
            pltpu.VMEM((CH, D), jnp.float32),
            pltpu.VMEM_SHARED((NPAD, D), jnp.float32),
            pltpu.SemaphoreType.DMA,
            pltpu.SemaphoreType.DMA,
        ],
    )
    return agg, deg


BN = 1024
GRID = NPAD // BN


def _tc_body(x_ref, p_ref, d_ref, wn_ref, ws_ref, b_ref, o_ref):
    p = p_ref[0] + p_ref[1]
    dsum = d_ref[0] + d_ref[1]
    deg = dsum[:, 0:1]
    mean = p * (1.0 / jnp.maximum(deg, 1.0))
    dn = (((1,), (1,)), ((), ()))
    hn = lax.dot_general(mean, wn_ref[...], dn,
                         preferred_element_type=jnp.float32)
    hs = lax.dot_general(x_ref[...], ws_ref[...], dn,
                         preferred_element_type=jnp.float32)
    o_ref[...] = hs + hn + b_ref[...]


_tc_layer = pl.pallas_call(
    _tc_body,
    grid=(GRID,),
    in_specs=[
        pl.BlockSpec((BN, D), lambda i: (i, 0)),
        pl.BlockSpec((NC, BN, D), lambda i: (0, i, 0)),
        pl.BlockSpec((NC, BN, D), lambda i: (0, i, 0)),
        pl.BlockSpec((D, D), lambda i: (0, 0)),
        pl.BlockSpec((D, D), lambda i: (0, 0)),
        pl.BlockSpec((1, D), lambda i: (0, 0)),
    ],
    out_specs=pl.BlockSpec((BN, D), lambda i: (i, 0)),
    out_shape=jax.ShapeDtypeStruct((NPAD, D), jnp.float32),
)


def kernel(h, edge_index, W_neigh1, W_self1, b_self1,
           W_neigh2, W_self2, b_self2):
    src = edge_index[0]
    dst = edge_index[1]
    pad = EPAD - E
    pad_dst = N + jnp.arange(pad, dtype=jnp.int32) % (NPAD - N)
    sd = jnp.concatenate([
        jnp.bitwise_or(src, jnp.left_shift(dst, 14)),
        jnp.left_shift(pad_dst, 14),
    ]).reshape(NW, NG, CH)
    x0 = jnp.pad(h, ((0, NPAD - N), (0, 0)))
    b1 = b_self1.reshape(1, D)
    b2 = b_self2.reshape(1, D)

    sc_agg, sc_deg = _sc_kernels()
    degp = sc_deg(sd)
    part1 = sc_agg(x0, sd)
    h1 = _tc_layer(x0, part1, degp, W_neigh1, W_self1, b1)
    part2 = sc_agg(h1, sd)
    h2 = _tc_layer(h1, part2, degp, W_neigh2, W_self2, b2)
    return h2[:N]
